# baseline (speedup 1.0000x reference)
import jax
import jax.numpy as jnp
from jax import lax
from jax.experimental import pallas as pl
from jax.experimental.pallas import tpu as pltpu

N = 8192
D = 64
S = 2
BM = 256

_DN_T = (((1,), (1,)), ((), ()))


def _make_adj_spec(s):
    return pl.BlockSpec((BM, N), lambda i, s=s: (S * i + s, 0))


def _fused_kernel(xs_ref, *rest):
    adj_refs = rest[:S]
    x_ref, w_ref, b_ref, o_ref = rest[S:]
    xv = x_ref[...].astype(jnp.bfloat16)
    w1 = w_ref[:, :D]
    w2 = w_ref[:, D:]
    bv = b_ref[...]
    for s in range(S):
        neigh = jnp.dot(adj_refs[s][...].astype(jnp.bfloat16), xv,
                        preferred_element_type=jnp.float32)
        acc = lax.dot_general(xs_ref[s * BM:(s + 1) * BM, :], w1, _DN_T,
                              preferred_element_type=jnp.float32)
        acc = acc + lax.dot_general(neigh, w2, _DN_T,
                                    preferred_element_type=jnp.float32)
        o_ref[s * BM:(s + 1) * BM, :] = jnp.maximum(acc + bv, 0.0)


@jax.jit
def kernel(x, adj_matrix, W, b):
    b2 = b.reshape(1, D)
    out = pl.pallas_call(
        _fused_kernel,
        grid=(N // (S * BM),),
        in_specs=[
            pl.BlockSpec((S * BM, D), lambda i: (i, 0)),
            *[_make_adj_spec(s) for s in range(S)],
            pl.BlockSpec((N, D), lambda i: (0, 0)),
            pl.BlockSpec((D, 2 * D), lambda i: (0, 0)),
            pl.BlockSpec((1, D), lambda i: (0, 0)),
        ],
        out_specs=pl.BlockSpec((S * BM, D), lambda i: (i, 0)),
        out_shape=jax.ShapeDtypeStruct((N, D), jnp.float32),
        compiler_params=pltpu.CompilerParams(
            dimension_semantics=(pltpu.PARALLEL,),
            vmem_limit_bytes=100 * 1024 * 1024,
        ),
    )(x, *([adj_matrix] * S), x, W, b2)
    return out


# speedup vs baseline: 1.0171x; 1.0171x over previous
import jax
import jax.numpy as jnp
from jax import lax
from jax.experimental import pallas as pl
from jax.experimental.pallas import tpu as pltpu

N = 8192
D = 64
S = 1
BM = 512

_DN_T = (((1,), (1,)), ((), ()))


def _make_adj_spec(s):
    return pl.BlockSpec((BM, N), lambda i, s=s: (S * i + s, 0))


def _fused_kernel(xs_ref, *rest):
    adj_refs = rest[:S]
    x_ref, w_ref, b_ref, o_ref = rest[S:]
    xv = x_ref[...].astype(jnp.bfloat16)
    w1 = w_ref[:, :D]
    w2 = w_ref[:, D:]
    bv = b_ref[...]
    for s in range(S):
        neigh = jnp.dot(adj_refs[s][...].astype(jnp.bfloat16), xv,
                        preferred_element_type=jnp.float32)
        acc = lax.dot_general(xs_ref[s * BM:(s + 1) * BM, :], w1, _DN_T,
                              preferred_element_type=jnp.float32)
        acc = acc + lax.dot_general(neigh, w2, _DN_T,
                                    preferred_element_type=jnp.float32)
        o_ref[s * BM:(s + 1) * BM, :] = jnp.maximum(acc + bv, 0.0)


@jax.jit
def kernel(x, adj_matrix, W, b):
    b2 = b.reshape(1, D)
    out = pl.pallas_call(
        _fused_kernel,
        grid=(N // (S * BM),),
        in_specs=[
            pl.BlockSpec((S * BM, D), lambda i: (i, 0)),
            *[_make_adj_spec(s) for s in range(S)],
            pl.BlockSpec((N, D), lambda i: (0, 0)),
            pl.BlockSpec((D, 2 * D), lambda i: (0, 0)),
            pl.BlockSpec((1, D), lambda i: (0, 0)),
        ],
        out_specs=pl.BlockSpec((S * BM, D), lambda i: (i, 0)),
        out_shape=jax.ShapeDtypeStruct((N, D), jnp.float32),
        compiler_params=pltpu.CompilerParams(
            dimension_semantics=(pltpu.PARALLEL,),
            vmem_limit_bytes=100 * 1024 * 1024,
        ),
    )(x, *([adj_matrix] * S), x, W, b2)
    return out
